# baseline (speedup 1.0000x reference)
import functools

import jax
import jax.numpy as jnp
from jax import lax
from jax.experimental import pallas as pl
from jax.experimental.pallas import tpu as pltpu
from jax.experimental.pallas import tpu_sc as plsc

NC = 2
NS = 16
NW = NC * NS
LANES = 16
NBUF = 4

_SC_PARAMS = dict(compiler_params=pltpu.CompilerParams(
    use_tc_tiling_on_sc=False, needs_layout_passes=False))


def _wid():
    return lax.axis_index("s") * NC + lax.axis_index("c")


@functools.lru_cache(maxsize=None)
def _build_transpose(V, D):
    TCH = 800
    n_ch = V // TCH
    t_per_w = -(-n_ch // NW)
    t_per_w += t_per_w % 2
    assert V % TCH == 0 and TCH % LANES == 0 and D == 2 * LANES

    mesh = plsc.VectorSubcoreMesh(core_axis_name="c", subcore_axis_name="s")

    @functools.partial(
        pl.kernel, mesh=mesh, **_SC_PARAMS,
        out_type=jax.ShapeDtypeStruct((V, D), jnp.float32),
        scratch_types=(
            [pltpu.VMEM((D, TCH), jnp.float32) for _ in range(2)]
            + [pltpu.VMEM((TCH, D), jnp.float32) for _ in range(2)]
            + [pltpu.SemaphoreType.DMA for _ in range(4)]
        ),
    )
    def tk(tcols, trows, cb0, cb1, rb0, rb1, gi0, gi1, go0, go1):
        cbs, rbs = (cb0, cb1), (rb0, rb1)
        gis, gos = (gi0, gi1), (go0, go1)
        w = _wid()
        riota = lax.iota(jnp.int32, 16)

        def chunk_of(t):
            return w + NW * t

        def in_copy(t, b):
            v0 = chunk_of(t) * TCH
            return pltpu.make_async_copy(
                tcols.at[:, pl.ds(v0, TCH)], cbs[b], gis[b])

        def out_copy(t, b):
            v0 = chunk_of(t) * TCH
            return pltpu.make_async_copy(
                rbs[b], trows.at[pl.ds(v0, TCH)], gos[b])

        @pl.when(chunk_of(0) < n_ch)
        def _():
            in_copy(0, 0).start()

        def pair(pi, _):
            for b in range(2):
                t = 2 * pi + b

                @pl.when(chunk_of(t) < n_ch)
                def _():
                    in_copy(t, b).wait()

                    @pl.when(chunk_of(t + 1) < n_ch)
                    def _():
                        in_copy(t + 1, 1 - b).start()

                    @pl.when(t >= 2)
                    def _():
                        out_copy(t - 2, b).wait()

                    dlo = lax.iota(jnp.int32, 16)
                    dhi = dlo + LANES

                    def vrow(v, _):
                        vs = lax.broadcast(v, (LANES,)).astype(jnp.int32)
                        rbs[b][v, pl.ds(0, LANES)] = plsc.load_gather(
                            cbs[b], [dlo, vs])
                        rbs[b][v, pl.ds(LANES, LANES)] = plsc.load_gather(
                            cbs[b], [dhi, vs])
                        return 0

                    lax.fori_loop(0, TCH, vrow, 0, unroll=8)
                    out_copy(t, b).start()
            return 0

        lax.fori_loop(0, t_per_w // 2, pair, 0)
        for t in (t_per_w - 2, t_per_w - 1):
            @pl.when(chunk_of(t) < n_ch)
            def _():
                out_copy(t, t % 2).wait()

    return tk


@functools.lru_cache(maxsize=None)
def _build_gather(B, L, V, D):
    Nb = B // NW
    Lc = 2
    n_ch = L // Lc
    assert B % NW == 0 and L % (Lc * NBUF) == 0 and D == 2 * LANES

    mesh = plsc.VectorSubcoreMesh(core_axis_name="c", subcore_axis_name="s")

    @functools.partial(
        pl.kernel, mesh=mesh, **_SC_PARAMS,
        out_type=jax.ShapeDtypeStruct((L, D, B), jnp.float32),
        scratch_types=(
            [pltpu.VMEM((Lc, Nb), jnp.int32) for _ in range(NBUF)]
            + [pltpu.VMEM((Lc, Nb, D), jnp.float32) for _ in range(NBUF)]
            + [pltpu.VMEM((Lc, D, Nb), jnp.float32) for _ in range(NBUF)]
            + [pltpu.VMEM((L, D), jnp.float32)]
            + [pltpu.SemaphoreType.DMA for _ in range(2 * NBUF)]
        ),
    )
    def gk(trows, xT, pos_hbm, out, *refs):
        idx_v = refs[0:NBUF]
        gbuf = refs[NBUF:2 * NBUF]
        tbuf = refs[2 * NBUF:3 * NBUF]
        pos_v = refs[3 * NBUF]
        gsem = refs[3 * NBUF + 1:3 * NBUF + 1 + NBUF]
        osem = refs[3 * NBUF + 1 + NBUF:3 * NBUF + 1 + 2 * NBUF]

        w = _wid()
        b0 = w * Nb
        pltpu.sync_copy(pos_hbm, pos_v)
        riota = lax.iota(jnp.int32, 16)

        def start_gathers(c, b):
            l0 = c * Lc
            for li in range(Lc):
                pltpu.sync_copy(xT.at[l0 + li, pl.ds(b0, Nb)],
                                idx_v[b].at[li])
            for li in range(Lc):
                pltpu.make_async_copy(
                    trows.at[idx_v[b].at[li]], gbuf[b].at[li],
                    gsem[b]).start()

        def wait_gathers(b):
            for li in range(Lc):
                pltpu.make_async_copy(
                    trows.at[idx_v[b].at[li]], gbuf[b].at[li],
                    gsem[b]).wait()

        def start_outs(c, b):
            l0 = c * Lc
            for li in range(Lc):
                pltpu.make_async_copy(
                    tbuf[b].at[li], out.at[l0 + li, :, pl.ds(b0, Nb)],
                    osem[b]).start()

        def wait_outs(c, b):
            l0 = c * Lc
            for li in range(Lc):
                pltpu.make_async_copy(
                    tbuf[b].at[li], out.at[l0 + li, :, pl.ds(b0, Nb)],
                    osem[b]).wait()

        start_gathers(0, 0)
        start_gathers(1, 1)

        def quad(pi, _):
            for b in range(NBUF):
                c = NBUF * pi + b
                wait_gathers(b)
                for li in range(Lc):
                    l = c * Lc + li
                    lsp = jnp.full((LANES,), li, jnp.int32)
                    lfull = lax.broadcast(l, (LANES,)).astype(jnp.int32)
                    for d in range(D):
                        dsp = jnp.full((LANES,), d, jnp.int32)
                        pd = plsc.load_gather(pos_v, [lfull, dsp])

                        def jblk(jb, _):
                            j0 = jb * LANES
                            vec = plsc.load_gather(
                                gbuf[b], [lsp, riota + j0, dsp]) + pd
                            tbuf[b][li, d, pl.ds(j0, LANES)] = vec
                            return 0

                        lax.fori_loop(0, Nb // LANES, jblk, 0, unroll=4)
                start_outs(c, b)
                b2 = (b + 2) % NBUF

                @pl.when(c >= 2)
                def _():
                    wait_outs(c - 2, b2)

                @pl.when(c + 2 < n_ch)
                def _():
                    start_gathers(c + 2, b2)
            return 0

        lax.fori_loop(0, n_ch // NBUF, quad, 0)
        for c in (n_ch - 2, n_ch - 1):
            wait_outs(c, c % NBUF)

    return gk


def kernel(x, token_table, pos_table):
    B, L = x.shape
    V, D = token_table.shape
    tk = _build_transpose(V, D)
    gk = _build_gather(B, L, V, D)
    t_cols = jnp.swapaxes(token_table, 0, 1)
    x_T = jnp.swapaxes(x, 0, 1)
    t_rows = tk(t_cols)
    out_phys = gk(t_rows, x_T, pos_table)
    return jnp.transpose(out_phys, (2, 0, 1))


# speedup vs baseline: 3.0923x; 3.0923x over previous
import functools

import jax
import jax.numpy as jnp
from jax import lax
from jax.experimental import pallas as pl
from jax.experimental.pallas import tpu as pltpu
from jax.experimental.pallas import tpu_sc as plsc

NC = 2
NS = 16
NW = NC * NS
LANES = 16
NBUF = 4

_SC_PARAMS = dict(compiler_params=pltpu.CompilerParams(
    use_tc_tiling_on_sc=False, needs_layout_passes=False))


@functools.lru_cache(maxsize=None)
def _build_gather(B, L, V, D):
    Nb = B // NW
    Lc = 2
    n_ch = L // Lc
    assert B % NW == 0 and L % (Lc * NBUF) == 0 and D == 2 * LANES
    assert Nb % LANES == 0

    mesh = plsc.VectorSubcoreMesh(core_axis_name="c", subcore_axis_name="s")

    @functools.partial(
        pl.kernel, mesh=mesh, **_SC_PARAMS,
        out_type=jax.ShapeDtypeStruct((L, D, B), jnp.float32),
        scratch_types=(
            [pltpu.VMEM((Nb,), jnp.int32) for _ in range(Lc * NBUF)]
            + [pltpu.VMEM((Nb, D), jnp.float32) for _ in range(Lc * NBUF)]
            + [pltpu.VMEM((D, Nb), jnp.float32) for _ in range(Lc * NBUF)]
            + [pltpu.VMEM((L, D), jnp.float32)]
            + [pltpu.SemaphoreType.DMA for _ in range(2 * NBUF)]
        ),
    )
    def gk(trows, xT, pos_hbm, out, *refs):
        nslot = Lc * NBUF
        idx_v = refs[0:nslot]
        gbuf = refs[nslot:2 * nslot]
        tbuf = refs[2 * nslot:3 * nslot]
        pos_v = refs[3 * nslot]
        gsem = refs[3 * nslot + 1:3 * nslot + 1 + NBUF]
        osem = refs[3 * nslot + 1 + NBUF:3 * nslot + 1 + 2 * NBUF]

        w = lax.axis_index("s") * NC + lax.axis_index("c")
        b0 = w * Nb
        pltpu.sync_copy(pos_hbm, pos_v)
        riota = lax.iota(jnp.int32, LANES)
        dzero = jnp.zeros((LANES,), jnp.int32)

        def start_gathers(c, b):
            l0 = c * Lc
            for li in range(Lc):
                pltpu.sync_copy(xT.at[l0 + li, pl.ds(b0, Nb)],
                                idx_v[Lc * b + li])
            for li in range(Lc):
                pltpu.make_async_copy(
                    trows.at[idx_v[Lc * b + li]], gbuf[Lc * b + li],
                    gsem[b]).start()

        def wait_gathers(b):
            for li in range(Lc):
                pltpu.make_async_copy(
                    trows.at[idx_v[Lc * b + li]], gbuf[Lc * b + li],
                    gsem[b]).wait()

        def start_outs(c, b):
            l0 = c * Lc
            for li in range(Lc):
                pltpu.make_async_copy(
                    tbuf[Lc * b + li], out.at[l0 + li, :, pl.ds(b0, Nb)],
                    osem[b]).start()

        def wait_outs(c, b):
            l0 = c * Lc
            for li in range(Lc):
                pltpu.make_async_copy(
                    tbuf[Lc * b + li], out.at[l0 + li, :, pl.ds(b0, Nb)],
                    osem[b]).wait()

        start_gathers(0, 0)
        start_gathers(1, 1)

        def quad(pi, _):
            for b in range(NBUF):
                c = NBUF * pi + b
                wait_gathers(b)
                for li in range(Lc):
                    l = c * Lc + li
                    lfull = lax.broadcast(l, (LANES,)).astype(jnp.int32)
                    g2 = gbuf[Lc * b + li]
                    t2 = tbuf[Lc * b + li]

                    def dloop(d, dsp):
                        pd = plsc.load_gather(pos_v, [lfull, dsp])

                        def jloop(jb, row):
                            vec = plsc.load_gather(g2, [row, dsp]) + pd
                            t2[d, pl.ds(jb * LANES, LANES)] = vec
                            return row + LANES

                        lax.fori_loop(0, Nb // LANES, jloop, riota, unroll=8)
                        return dsp + 1

                    lax.fori_loop(0, D, dloop, dzero, unroll=2)
                start_outs(c, b)
                b2 = (b + 2) % NBUF

                @pl.when(c >= 2)
                def _():
                    wait_outs(c - 2, b2)

                @pl.when(c + 2 < n_ch)
                def _():
                    start_gathers(c + 2, b2)
            return 0

        lax.fori_loop(0, n_ch // NBUF, quad, 0)
        for c in (n_ch - 2, n_ch - 1):
            wait_outs(c, c % NBUF)

    return gk


def kernel(x, token_table, pos_table):
    B, L = x.shape
    V, D = token_table.shape
    gk = _build_gather(B, L, V, D)
    x_T = jnp.swapaxes(x, 0, 1)
    out_phys = gk(token_table, x_T, pos_table)
    return jnp.transpose(out_phys, (2, 0, 1))


# speedup vs baseline: 5.4157x; 1.7514x over previous
import functools

import jax
import jax.numpy as jnp
from jax import lax
from jax.experimental import pallas as pl
from jax.experimental.pallas import tpu as pltpu
from jax.experimental.pallas import tpu_sc as plsc

NC = 2
NS = 16
NW = NC * NS
LANES = 16
NBUF = 4

_SC_PARAMS = dict(compiler_params=pltpu.CompilerParams(
    use_tc_tiling_on_sc=False, needs_layout_passes=False))


@functools.lru_cache(maxsize=None)
def _build_gather(B, L, V, D):
    Nb = B // NW
    Lc = 2
    n_ch = L // Lc
    assert B % NW == 0 and L % (Lc * NBUF) == 0 and D == 2 * LANES
    assert Nb % LANES == 0

    mesh = plsc.VectorSubcoreMesh(core_axis_name="c", subcore_axis_name="s")

    @functools.partial(
        pl.kernel, mesh=mesh, **_SC_PARAMS,
        out_type=jax.ShapeDtypeStruct((L, D, B), jnp.float32),
        scratch_types=(
            [pltpu.VMEM((L, Nb), jnp.int32)]
            + [pltpu.VMEM((Nb, D), jnp.float32) for _ in range(Lc * NBUF)]
            + [pltpu.VMEM((D, Nb + 1), jnp.float32) for _ in range(Lc * NBUF)]
            + [pltpu.VMEM((L, D), jnp.float32)]
            + [pltpu.SemaphoreType.DMA for _ in range(2 * NBUF)]
        ),
    )
    def gk(trows, xT, pos_hbm, out, *refs):
        nslot = Lc * NBUF
        idx_all = refs[0]
        gbuf = refs[1:1 + nslot]
        tbuf = refs[1 + nslot:1 + 2 * nslot]
        pos_v = refs[1 + 2 * nslot]
        gsem = refs[2 + 2 * nslot:2 + 2 * nslot + NBUF]
        osem = refs[2 + 2 * nslot + NBUF:2 + 2 * nslot + 2 * NBUF]

        w = lax.axis_index("s") * NC + lax.axis_index("c")
        b0 = w * Nb
        pltpu.sync_copy(xT.at[:, pl.ds(b0, Nb)], idx_all)
        pltpu.sync_copy(pos_hbm, pos_v)
        dlo = lax.iota(jnp.int32, LANES)
        dhi = dlo + LANES

        def start_gathers(c, b):
            l0 = c * Lc
            for li in range(Lc):
                pltpu.make_async_copy(
                    trows.at[idx_all.at[l0 + li]], gbuf[Lc * b + li],
                    gsem[b]).start()

        def wait_gathers(c, b):
            l0 = c * Lc
            for li in range(Lc):
                pltpu.make_async_copy(
                    trows.at[idx_all.at[l0 + li]], gbuf[Lc * b + li],
                    gsem[b]).wait()

        def start_outs(c, b):
            l0 = c * Lc
            for li in range(Lc):
                pltpu.make_async_copy(
                    tbuf[Lc * b + li].at[:, pl.ds(0, Nb)],
                    out.at[l0 + li, :, pl.ds(b0, Nb)],
                    osem[b]).start()

        def wait_outs(c, b):
            l0 = c * Lc
            for li in range(Lc):
                pltpu.make_async_copy(
                    tbuf[Lc * b + li].at[:, pl.ds(0, Nb)],
                    out.at[l0 + li, :, pl.ds(b0, Nb)],
                    osem[b]).wait()

        start_gathers(0, 0)
        start_gathers(1, 1)

        def quad(pi, _):
            for b in range(NBUF):
                c = NBUF * pi + b
                wait_gathers(c, b)
                for li in range(Lc):
                    l = c * Lc + li
                    p0 = pos_v[l, pl.ds(0, LANES)]
                    p1 = pos_v[l, pl.ds(LANES, LANES)]
                    g2 = gbuf[Lc * b + li]
                    t2 = tbuf[Lc * b + li]

                    def jloop(j, _):
                        v0 = g2[j, pl.ds(0, LANES)] + p0
                        v1 = g2[j, pl.ds(LANES, LANES)] + p1
                        jsp = lax.broadcast(j, (LANES,)).astype(jnp.int32)
                        plsc.store_scatter(t2, [dlo, jsp], v0)
                        plsc.store_scatter(t2, [dhi, jsp], v1)
                        return 0

                    lax.fori_loop(0, Nb, jloop, 0, unroll=8)
                start_outs(c, b)
                b2 = (b + 2) % NBUF

                @pl.when(c >= 2)
                def _():
                    wait_outs(c - 2, b2)

                @pl.when(c + 2 < n_ch)
                def _():
                    start_gathers(c + 2, b2)
            return 0

        lax.fori_loop(0, n_ch // NBUF, quad, 0)
        for c in (n_ch - 2, n_ch - 1):
            wait_outs(c, c % NBUF)

    return gk


def kernel(x, token_table, pos_table):
    B, L = x.shape
    V, D = token_table.shape
    gk = _build_gather(B, L, V, D)
    x_T = jnp.swapaxes(x, 0, 1)
    out_phys = gk(token_table, x_T, pos_table)
    return jnp.transpose(out_phys, (2, 0, 1))


# speedup vs baseline: 6.3541x; 1.1733x over previous
import functools

import jax
import jax.numpy as jnp
from jax import lax
from jax.experimental import pallas as pl
from jax.experimental.pallas import tpu as pltpu
from jax.experimental.pallas import tpu_sc as plsc

NC = 2
NS = 16
NW = NC * NS
LANES = 16
NBUF = 4

_SC_PARAMS = dict(compiler_params=pltpu.CompilerParams(
    use_tc_tiling_on_sc=False, needs_layout_passes=False))


@functools.lru_cache(maxsize=None)
def _build_gather(B, L, V, D):
    Nb = B // NW
    Lc = 2
    n_ch = L // Lc
    assert B % NW == 0 and L % (Lc * NBUF) == 0 and D == 2 * LANES
    assert Nb % LANES == 0

    mesh = plsc.VectorSubcoreMesh(core_axis_name="c", subcore_axis_name="s")

    @functools.partial(
        pl.kernel, mesh=mesh, **_SC_PARAMS,
        out_type=jax.ShapeDtypeStruct((L, D // 8, B // 128, 8, 128),
                                      jnp.float32),
        scratch_types=(
            [pltpu.VMEM((L, Nb), jnp.int32)]
            + [pltpu.VMEM((Nb, D), jnp.float32) for _ in range(Lc * NBUF)]
            + [pltpu.VMEM((D // 8, 8, Nb + 1), jnp.float32)
               for _ in range(Lc * NBUF)]
            + [pltpu.VMEM((L, D), jnp.float32)]
            + [pltpu.SemaphoreType.DMA for _ in range(2 * NBUF)]
        ),
    )
    def gk(trows, xT, pos_hbm, out, *refs):
        nslot = Lc * NBUF
        idx_all = refs[0]
        gbuf = refs[1:1 + nslot]
        tbuf = refs[1 + nslot:1 + 2 * nslot]
        pos_v = refs[1 + 2 * nslot]
        gsem = refs[2 + 2 * nslot:2 + 2 * nslot + NBUF]
        osem = refs[2 + 2 * nslot + NBUF:2 + 2 * nslot + 2 * NBUF]

        w = lax.axis_index("s") * NC + lax.axis_index("c")
        b0 = w * Nb
        pltpu.sync_copy(xT.at[:, pl.ds(b0, Nb)], idx_all)
        pltpu.sync_copy(pos_hbm, pos_v)
        dlo = lax.iota(jnp.int32, LANES)
        dhi = dlo + LANES
        dh_lo, dl_lo = dlo // 8, dlo % 8
        dh_hi, dl_hi = dhi // 8, dhi % 8

        def start_gathers(c, b):
            l0 = c * Lc
            for li in range(Lc):
                pltpu.make_async_copy(
                    trows.at[idx_all.at[l0 + li]], gbuf[Lc * b + li],
                    gsem[b]).start()

        def wait_gathers(c, b):
            l0 = c * Lc
            for li in range(Lc):
                pltpu.make_async_copy(
                    trows.at[idx_all.at[l0 + li]], gbuf[Lc * b + li],
                    gsem[b]).wait()

        def start_outs(c, b):
            l0 = c * Lc
            for li in range(Lc):
                pltpu.make_async_copy(
                    tbuf[Lc * b + li].at[:, :, pl.ds(0, Nb)],
                    out.at[l0 + li, :, w, :, :],
                    osem[b]).start()

        def wait_outs(c, b):
            l0 = c * Lc
            for li in range(Lc):
                pltpu.make_async_copy(
                    tbuf[Lc * b + li].at[:, :, pl.ds(0, Nb)],
                    out.at[l0 + li, :, w, :, :],
                    osem[b]).wait()

        start_gathers(0, 0)
        start_gathers(1, 1)

        def quad(pi, _):
            for b in range(NBUF):
                c = NBUF * pi + b
                wait_gathers(c, b)
                for li in range(Lc):
                    l = c * Lc + li
                    p0 = pos_v[l, pl.ds(0, LANES)]
                    p1 = pos_v[l, pl.ds(LANES, LANES)]
                    g2 = gbuf[Lc * b + li]
                    t2 = tbuf[Lc * b + li]

                    def jloop(j, _):
                        v0 = g2[j, pl.ds(0, LANES)] + p0
                        v1 = g2[j, pl.ds(LANES, LANES)] + p1
                        jsp = lax.broadcast(j, (LANES,)).astype(jnp.int32)
                        plsc.store_scatter(t2, [dh_lo, dl_lo, jsp], v0)
                        plsc.store_scatter(t2, [dh_hi, dl_hi, jsp], v1)
                        return 0

                    lax.fori_loop(0, Nb, jloop, 0, unroll=8)
                start_outs(c, b)
                b2 = (b + 2) % NBUF

                @pl.when(c >= 2)
                def _():
                    wait_outs(c - 2, b2)

                @pl.when(c + 2 < n_ch)
                def _():
                    start_gathers(c + 2, b2)
            return 0

        lax.fori_loop(0, n_ch // NBUF, quad, 0)
        for c in (n_ch - 2, n_ch - 1):
            wait_outs(c, c % NBUF)

    return gk


def kernel(x, token_table, pos_table):
    B, L = x.shape
    V, D = token_table.shape
    gk = _build_gather(B, L, V, D)
    x_T = jnp.swapaxes(x, 0, 1)
    out5 = gk(token_table, x_T, pos_table)
    return jnp.transpose(out5, (2, 4, 0, 1, 3)).reshape(B, L, D)
